# baseline (speedup 1.0000x reference)
import functools

import jax
import jax.numpy as jnp
from jax import lax
from jax.experimental import pallas as pl
from jax.experimental.pallas import tpu as pltpu
from jax.experimental.pallas import tpu_sc as plsc

_S = 512


def _tc_body(coord_ref, rois_ref, slot_ref, cnt_ref, par_ref):
    f32 = jnp.float32
    i32 = jnp.int32
    x = coord_ref[0, 0]
    y = coord_ref[0, 1]
    z = coord_ref[0, 2]
    cx = rois_ref[0, 0, 0, 1]
    cy = rois_ref[0, 0, 0, 2]
    cz = rois_ref[0, 0, 0, 3]
    dx = rois_ref[0, 0, 0, 4]
    dy = rois_ref[0, 0, 0, 5]
    dz = rois_ref[0, 0, 0, 6]
    rz = rois_ref[0, 0, 0, 7]
    cosa = jnp.cos(rz)
    sina = -jnp.sin(rz)
    sx = x - cx
    sy = y - cy
    lx = sx * cosa - sy * sina
    ly = sx * sina + sy * cosa
    mask = ((jnp.abs(z - (cz + 0.5 * dz)) <= 0.5 * dz)
            & (jnp.abs(lx) < 0.5 * dx)
            & (jnp.abs(ly) < 0.5 * dy))
    mf = mask.astype(f32)
    ii = lax.broadcasted_iota(i32, (128, 128), 0)
    jj = lax.broadcasted_iota(i32, (128, 128), 1)
    upper_inc = (ii <= jj).astype(f32)
    lower_strict = (jj < ii).astype(f32)
    csw = jnp.dot(mf, upper_inc, preferred_element_type=f32)
    rowtot = csw[:, 127:128]
    off = jnp.dot(lower_strict, rowtot, preferred_element_type=f32)
    cs = csw + off
    slot = jnp.where(mask & (cs <= float(_S)), cs - 1.0, -1.0).astype(i32)
    slot_ref[0, 0] = slot
    cnt = jnp.sum(mf).astype(i32)
    cnt_ref[0, 0, 0] = jnp.broadcast_to(cnt, (16,))
    par_ref[0, 0, 0] = jnp.broadcast_to(cosa, (16,))
    par_ref[0, 0, 1] = jnp.broadcast_to(sina, (16,))
    par_ref[0, 0, 2] = jnp.broadcast_to(cx, (16,))
    par_ref[0, 0, 3] = jnp.broadcast_to(cy, (16,))
    par_ref[0, 0, 4] = jnp.broadcast_to(cz, (16,))


def _sc_body(n_pts, n_pad, m_per_b, bpw,
             slots_hbm, cnts_hbm, params_hbm, planes_hbm, feats_hbm,
             out_ft, out_xyz,
             xpl, ypl, zpl, sbuf, compact, selg, cbuf, pbuf,
             xb, yb, zb, rows0, rows1, sem0, sem1):
    i32 = jnp.int32
    info = plsc.get_sparse_core_info()
    nc = info.num_cores
    wid = lax.axis_index("s") * nc + lax.axis_index("c")
    base_box = wid * bpw
    batch = base_box // m_per_b

    pbase = batch * 3 * n_pad
    pltpu.sync_copy(planes_hbm.at[pl.ds(pbase, n_pad)], xpl)
    pltpu.sync_copy(planes_hbm.at[pl.ds(pbase + n_pad, n_pad)], ypl)
    pltpu.sync_copy(planes_hbm.at[pl.ds(pbase + 2 * n_pad, n_pad)], zpl)

    lanes = lax.iota(i32, 16)

    def one_box(i, _):
        b = base_box + i
        pltpu.sync_copy(slots_hbm.at[pl.ds(b * n_pts, n_pts)], sbuf)
        pltpu.sync_copy(cnts_hbm.at[pl.ds(b * 16, 16)], cbuf)
        pltpu.sync_copy(params_hbm.at[pl.ds(b * 96, 96)], pbuf)
        cntv = cbuf[...]
        cosa = pbuf[pl.ds(0, 16)]
        sina = pbuf[pl.ds(16, 16)]
        cxv = pbuf[pl.ds(32, 16)]
        cyv = pbuf[pl.ds(48, 16)]
        czv = pbuf[pl.ds(64, 16)]
        rbv = pbuf[pl.ds(80, 16)].astype(i32)
        nonempty = cntv > 0
        cxo = jnp.where(nonempty, cxv, 0.0)
        cyo = jnp.where(nonempty, cyv, 0.0)
        czo = jnp.where(nonempty, czv, 0.0)
        safev = jnp.maximum(cntv, 1)

        def scatter_pts(t, idxv):
            svs = [sbuf[pl.ds(t * 64 + u * 16, 16)] for u in range(4)]
            mx = jnp.maximum(jnp.maximum(svs[0], svs[1]),
                             jnp.maximum(svs[2], svs[3]))
            any_sel = jnp.any(mx >= 0)

            @pl.when(any_sel)
            def _do():
                iv = idxv
                for u in range(4):
                    plsc.store_scatter(compact, [svs[u]], iv, mask=svs[u] >= 0)
                    iv = iv + 16

            return idxv + 64

        lax.fori_loop(0, n_pts // 64, scatter_pts, lanes)

        def one_chunk(jc, jv):
            sl16 = pl.ds(jc * 16, 16)
            pos = jnp.where(jv < cntv, jv, jv % safev)
            sl = plsc.load_gather(compact, [pos])
            sl = jnp.where(nonempty, sl, n_pts)
            selg[sl16] = sl + rbv
            xg = plsc.load_gather(xpl, [sl])
            yg = plsc.load_gather(ypl, [sl])
            zg = plsc.load_gather(zpl, [sl])
            gx = xg - cxo
            gy = yg - cyo
            xb[sl16] = gx * cosa - gy * sina
            yb[sl16] = gx * sina + gy * cosa
            zb[sl16] = zg - czo
            return jv + 16

        lax.fori_loop(0, _S // 16, one_chunk, lanes)

        obase = b * 3 * _S
        pltpu.sync_copy(xb, out_xyz.at[pl.ds(obase, _S)])
        pltpu.sync_copy(yb, out_xyz.at[pl.ds(obase + _S, _S)])
        pltpu.sync_copy(zb, out_xyz.at[pl.ds(obase + 2 * _S, _S)])

        rbufs = (rows0, rows1)
        sems = (sem0, sem1)
        cps = []
        for c in range(_S // 128):
            cp = pltpu.async_copy(feats_hbm.at[selg.at[pl.ds(c * 128, 128)]],
                                  rbufs[c % 2], sems[c % 2])
            cps.append(cp)
            if c >= 1:
                cps[c - 1].wait()
                pltpu.sync_copy(rbufs[(c - 1) % 2],
                                out_ft.at[pl.ds(b * _S + (c - 1) * 128, 128)])
        cps[-1].wait()
        c_last = _S // 128 - 1
        pltpu.sync_copy(rbufs[c_last % 2],
                        out_ft.at[pl.ds(b * _S + c_last * 128, 128)])
        return 0

    lax.fori_loop(0, bpw, one_box, 0)


def kernel(feats, coordinate, batch_inds, rois):
    f32 = jnp.float32
    i32 = jnp.int32
    B, N, C = feats.shape
    BM = rois.shape[0]
    M = BM // B
    NR = 128

    coord4 = coordinate.astype(f32).transpose(0, 2, 1).reshape(B, 3, NR, NR)
    rois4 = rois.astype(f32).reshape(B, M, 1, 8)
    slot_out, cnt_out, par_out = pl.pallas_call(
        _tc_body,
        grid=(B, M),
        in_specs=[
            pl.BlockSpec((1, 3, NR, NR), lambda b, m: (b, 0, 0, 0)),
            pl.BlockSpec((1, 1, 1, 8), lambda b, m: (b, m, 0, 0)),
        ],
        out_specs=[
            pl.BlockSpec((1, 1, NR, NR), lambda b, m: (b, m, 0, 0)),
            pl.BlockSpec((1, 1, 1, 16), lambda b, m: (b, m, 0, 0)),
            pl.BlockSpec((1, 1, 6, 16), lambda b, m: (b, m, 0, 0)),
        ],
        out_shape=[
            jax.ShapeDtypeStruct((B, M, NR, NR), i32),
            jax.ShapeDtypeStruct((B, M, 1, 16), i32),
            jax.ShapeDtypeStruct((B, M, 6, 16), f32),
        ],
    )(coord4, rois4)

    n_pad = 16 * ((N // 16) + 1)
    coord_t = coordinate.astype(f32).transpose(0, 2, 1)
    planes = jnp.pad(coord_t, ((0, 0), (0, 0), (0, n_pad - N)))
    planes = planes.reshape(B * 3 * n_pad)
    feats_tab = jnp.pad(feats.astype(f32), ((0, 0), (0, 1), (0, 0)))
    feats_tab = feats_tab.reshape(B * (N + 1), C)

    slots = slot_out.reshape(BM * N)
    cnts = cnt_out.reshape(BM * 16)
    row_base = (jnp.arange(BM, dtype=i32) // M) * (N + 1)
    rbase16 = jnp.broadcast_to(row_base[:, None, None].astype(f32),
                               (BM, 1, 16))
    params = jnp.concatenate([par_out.reshape(BM, 6, 16)[:, :5],
                              rbase16], axis=1).reshape(BM * 96)

    info = plsc.get_sparse_core_info()
    nw = info.num_cores * info.num_subcores
    bpw = BM // nw

    mesh = plsc.VectorSubcoreMesh(core_axis_name="c", subcore_axis_name="s")
    body = functools.partial(_sc_body, N, n_pad, M, bpw)
    run = pl.kernel(
        body,
        mesh=mesh,
        compiler_params=pltpu.CompilerParams(needs_layout_passes=False, use_tc_tiling_on_sc=False),
        out_type=[
            jax.ShapeDtypeStruct((BM * _S, C), f32),
            jax.ShapeDtypeStruct((BM * 3 * _S,), f32),
        ],
        scratch_types=[
            pltpu.VMEM((n_pad,), f32),
            pltpu.VMEM((n_pad,), f32),
            pltpu.VMEM((n_pad,), f32),
            pltpu.VMEM((N,), i32),
            pltpu.VMEM((_S,), i32),
            pltpu.VMEM((_S,), i32),
            pltpu.VMEM((16,), i32),
            pltpu.VMEM((96,), f32),
            pltpu.VMEM((_S,), f32),
            pltpu.VMEM((_S,), f32),
            pltpu.VMEM((_S,), f32),
            pltpu.VMEM((128, C), f32),
            pltpu.VMEM((128, C), f32),
            pltpu.SemaphoreType.DMA,
            pltpu.SemaphoreType.DMA,
        ],
    )
    out_ft, out_xyz = run(slots, cnts, params, planes, feats_tab)
    out_ft = out_ft.reshape(BM, _S, C)
    out_xyz = out_xyz.reshape(BM, 3, _S).transpose(0, 2, 1)
    return jnp.concatenate([out_xyz, out_ft], axis=-1)


# speedup vs baseline: 1.0285x; 1.0285x over previous
import functools

import jax
import jax.numpy as jnp
from jax import lax
from jax.experimental import pallas as pl
from jax.experimental.pallas import tpu as pltpu
from jax.experimental.pallas import tpu_sc as plsc

_S = 512


def _tc_body(coord_ref, rois_ref, slot_ref, par_ref):
    f32 = jnp.float32
    i32 = jnp.int32
    x = coord_ref[0, 0]
    y = coord_ref[0, 1]
    z = coord_ref[0, 2]
    cx = rois_ref[0, 0, 0, 1]
    cy = rois_ref[0, 0, 0, 2]
    cz = rois_ref[0, 0, 0, 3]
    dx = rois_ref[0, 0, 0, 4]
    dy = rois_ref[0, 0, 0, 5]
    dz = rois_ref[0, 0, 0, 6]
    rz = rois_ref[0, 0, 0, 7]
    cosa = jnp.cos(rz)
    sina = -jnp.sin(rz)
    sx = x - cx
    sy = y - cy
    lx = sx * cosa - sy * sina
    ly = sx * sina + sy * cosa
    mask = ((jnp.abs(z - (cz + 0.5 * dz)) <= 0.5 * dz)
            & (jnp.abs(lx) < 0.5 * dx)
            & (jnp.abs(ly) < 0.5 * dy))
    mf = mask.astype(f32)
    ii = lax.broadcasted_iota(i32, (128, 128), 0)
    jj = lax.broadcasted_iota(i32, (128, 128), 1)
    upper_inc = (ii <= jj).astype(f32)
    lower_strict = (jj < ii).astype(f32)
    csw = jnp.dot(mf, upper_inc, preferred_element_type=f32)
    rowtot = csw[:, 127:128]
    off = jnp.dot(lower_strict, rowtot, preferred_element_type=f32)
    cs = csw + off
    slot = jnp.where(mask & (cs <= float(_S)), cs - 1.0, -1.0).astype(i32)
    slot_ref[0, 0] = slot
    cnt = jnp.sum(mf)
    par_ref[0, 0, 5] = jnp.broadcast_to(cnt, (16,))
    par_ref[0, 0, 0] = jnp.broadcast_to(cosa, (16,))
    par_ref[0, 0, 1] = jnp.broadcast_to(sina, (16,))
    par_ref[0, 0, 2] = jnp.broadcast_to(cx, (16,))
    par_ref[0, 0, 3] = jnp.broadcast_to(cy, (16,))
    par_ref[0, 0, 4] = jnp.broadcast_to(cz, (16,))


def _sc_body(n_pts, n_pad, m_per_b, bpw,
             slots_hbm, params_hbm, planes_hbm, feats_hbm,
             out_ft, out_xyz,
             xpl, ypl, zpl, sbuf0, sbuf1, compact, selg, pbuf,
             xb, yb, zb, rows0, rows1, sem0, sem1, ssem0, ssem1):
    i32 = jnp.int32
    info = plsc.get_sparse_core_info()
    nc = info.num_cores
    wid = lax.axis_index("s") * nc + lax.axis_index("c")
    base_box = wid * bpw
    batch = base_box // m_per_b

    pbase = batch * 3 * n_pad
    pltpu.sync_copy(planes_hbm.at[pl.ds(pbase, n_pad)], xpl)
    pltpu.sync_copy(planes_hbm.at[pl.ds(pbase + n_pad, n_pad)], ypl)
    pltpu.sync_copy(planes_hbm.at[pl.ds(pbase + 2 * n_pad, n_pad)], zpl)

    lanes = lax.iota(i32, 16)
    sbufs = (sbuf0, sbuf1)
    ssems = (ssem0, ssem1)
    scp0 = pltpu.async_copy(slots_hbm.at[pl.ds(base_box * n_pts, n_pts)],
                            sbuf0, ssem0)
    pending = [scp0]

    def one_box(i, b, sbuf):
        pltpu.sync_copy(params_hbm.at[pl.ds(b * 128, 128)], pbuf)
        cosa = pbuf[pl.ds(0, 16)]
        sina = pbuf[pl.ds(16, 16)]
        cxv = pbuf[pl.ds(32, 16)]
        cyv = pbuf[pl.ds(48, 16)]
        czv = pbuf[pl.ds(64, 16)]
        cntv = pbuf[pl.ds(80, 16)].astype(i32)
        rbv = pbuf[pl.ds(96, 16)].astype(i32)
        nonempty = cntv > 0
        cxo = jnp.where(nonempty, cxv, 0.0)
        cyo = jnp.where(nonempty, cyv, 0.0)
        czo = jnp.where(nonempty, czv, 0.0)
        safev = jnp.maximum(cntv, 1)

        def scatter_pts(t, idxv):
            for u in range(4):
                sv = sbuf[pl.ds(t * 64 + u * 16, 16)]
                plsc.store_scatter(compact, [sv], idxv, mask=sv >= 0)
                idxv = idxv + 16
            return idxv

        lax.fori_loop(0, n_pts // 64, scatter_pts, lanes)

        def one_chunk(jc, jv):
            sl16 = pl.ds(jc * 16, 16)
            pos = jnp.where(jv < cntv, jv, jv % safev)
            sl = plsc.load_gather(compact, [pos])
            sl = jnp.where(nonempty, sl, n_pts)
            selg[sl16] = sl + rbv
            xg = plsc.load_gather(xpl, [sl])
            yg = plsc.load_gather(ypl, [sl])
            zg = plsc.load_gather(zpl, [sl])
            gx = xg - cxo
            gy = yg - cyo
            xb[sl16] = gx * cosa - gy * sina
            yb[sl16] = gx * sina + gy * cosa
            zb[sl16] = zg - czo
            return jv + 16

        lax.fori_loop(0, _S // 16, one_chunk, lanes)


        obase = b * 3 * _S
        pltpu.sync_copy(xb, out_xyz.at[pl.ds(obase, _S)])
        pltpu.sync_copy(yb, out_xyz.at[pl.ds(obase + _S, _S)])
        pltpu.sync_copy(zb, out_xyz.at[pl.ds(obase + 2 * _S, _S)])

        rbufs = (rows0, rows1)
        sems = (sem0, sem1)
        cps = []
        for c in range(_S // 128):
            cp = pltpu.async_copy(feats_hbm.at[selg.at[pl.ds(c * 128, 128)]],
                                  rbufs[c % 2], sems[c % 2])
            cps.append(cp)
            if c >= 1:
                cps[c - 1].wait()
                pltpu.sync_copy(rbufs[(c - 1) % 2],
                                out_ft.at[pl.ds(b * _S + (c - 1) * 128, 128)])
        cps[-1].wait()
        c_last = _S // 128 - 1
        pltpu.sync_copy(rbufs[c_last % 2],
                        out_ft.at[pl.ds(b * _S + c_last * 128, 128)])

    for i in range(bpw):
        if i + 1 < bpw:
            pending.append(pltpu.async_copy(
                slots_hbm.at[pl.ds((base_box + i + 1) * n_pts, n_pts)],
                sbufs[(i + 1) % 2], ssems[(i + 1) % 2]))
        pending[i].wait()
        one_box(i, base_box + i, sbufs[i % 2])


def kernel(feats, coordinate, batch_inds, rois):
    f32 = jnp.float32
    i32 = jnp.int32
    B, N, C = feats.shape
    BM = rois.shape[0]
    M = BM // B
    NR = 128

    coord4 = coordinate.astype(f32).transpose(0, 2, 1).reshape(B, 3, NR, NR)
    rois4 = rois.astype(f32).reshape(B, M, 1, 8)
    slot_out, par_out = pl.pallas_call(
        _tc_body,
        grid=(B, M),
        in_specs=[
            pl.BlockSpec((1, 3, NR, NR), lambda b, m: (b, 0, 0, 0)),
            pl.BlockSpec((1, 1, 1, 8), lambda b, m: (b, m, 0, 0)),
        ],
        out_specs=[
            pl.BlockSpec((1, 1, NR, NR), lambda b, m: (b, m, 0, 0)),
            pl.BlockSpec((1, 1, 6, 16), lambda b, m: (b, m, 0, 0)),
        ],
        out_shape=[
            jax.ShapeDtypeStruct((B, M, NR, NR), i32),
            jax.ShapeDtypeStruct((B, M, 6, 16), f32),
        ],
    )(coord4, rois4)

    n_pad = 16 * ((N // 16) + 1)
    coord_t = coordinate.astype(f32).transpose(0, 2, 1)
    planes = jnp.pad(coord_t, ((0, 0), (0, 0), (0, n_pad - N)))
    planes = planes.reshape(B * 3 * n_pad)
    feats_tab = jnp.pad(feats.astype(f32), ((0, 0), (0, 1), (0, 0)))
    feats_tab = feats_tab.reshape(B * (N + 1), C)

    slots = slot_out.reshape(BM * N)
    row_base = (jnp.arange(BM, dtype=i32) // M) * (N + 1)
    rbase16 = jnp.broadcast_to(row_base[:, None, None].astype(f32),
                               (BM, 1, 16))
    params = jnp.concatenate([par_out.reshape(BM, 6, 16), rbase16,
                              jnp.zeros((BM, 1, 16), f32)],
                             axis=1).reshape(BM * 128)

    info = plsc.get_sparse_core_info()
    nw = info.num_cores * info.num_subcores
    bpw = BM // nw

    mesh = plsc.VectorSubcoreMesh(core_axis_name="c", subcore_axis_name="s")
    body = functools.partial(_sc_body, N, n_pad, M, bpw)
    run = pl.kernel(
        body,
        mesh=mesh,
        compiler_params=pltpu.CompilerParams(needs_layout_passes=False, use_tc_tiling_on_sc=False),
        out_type=[
            jax.ShapeDtypeStruct((BM * _S, C), f32),
            jax.ShapeDtypeStruct((BM * 3 * _S,), f32),
        ],
        scratch_types=[
            pltpu.VMEM((n_pad,), f32),
            pltpu.VMEM((n_pad,), f32),
            pltpu.VMEM((n_pad,), f32),
            pltpu.VMEM((N,), i32),
            pltpu.VMEM((N,), i32),
            pltpu.VMEM((_S,), i32),
            pltpu.VMEM((_S,), i32),
            pltpu.VMEM((128,), f32),
            pltpu.VMEM((_S,), f32),
            pltpu.VMEM((_S,), f32),
            pltpu.VMEM((_S,), f32),
            pltpu.VMEM((128, C), f32),
            pltpu.VMEM((128, C), f32),
            pltpu.SemaphoreType.DMA,
            pltpu.SemaphoreType.DMA,
            pltpu.SemaphoreType.DMA,
            pltpu.SemaphoreType.DMA,
        ],
    )
    out_ft, out_xyz = run(slots, params, planes, feats_tab)
    out_ft = out_ft.reshape(BM, _S, C)
    out_xyz = out_xyz.reshape(BM, 3, _S).transpose(0, 2, 1)
    return jnp.concatenate([out_xyz, out_ft], axis=-1)
